# Initial kernel scaffold; baseline (speedup 1.0000x reference)
import jax
import jax.numpy as jnp
from jax.experimental import pallas as pl


def kernel(X, edge_idx, C):
    raise NotImplementedError("write your pallas kernel here")


# speedup vs baseline: 1.9894x; 1.9894x over previous
import functools

import jax
import jax.numpy as jnp
from jax import lax
from jax.experimental import pallas as pl
from jax.experimental.pallas import tpu as pltpu
from jax.experimental.pallas import tpu_sc as plsc

_NC = 2
_NS = 16
_NW = _NC * _NS
_L = 16

_N = 10000
_K = 64
_D = 12

_CHUNK = 1024
_NCHUNK = _N * _K // _CHUNK
_ROWS = 2 * _CHUNK
_NSUB = _ROWS // 128


def _edge2grids_body(x_hbm, e_hbm, c_hbm, out_hbm, mout_hbm,
                     c_v, m_v, e_v, idx_v, rows_v, mk_v, sem):
    wid = lax.axis_index("s") * _NC + lax.axis_index("c")
    iota = lax.iota(jnp.int32, _L)

    pltpu.sync_copy(c_hbm, c_v)

    def mask_tbl(i, carry):
        off = i * _L
        cv = c_v[pl.ds(off, _L)]
        m_v[pl.ds(off, _L)] = jnp.where(cv > 0, 1.0, 0.0).astype(jnp.float32)
        return carry

    lax.fori_loop(0, _N // _L, mask_tbl, 0)

    def chunk_body(i, carry):
        c = wid + i * _NW

        @pl.when(c < _NCHUNK)
        def _():
            base = c * _CHUNK
            pltpu.sync_copy(e_hbm.at[pl.ds(base, _CHUNK)], e_v)

            def grp(g, carry2):
                off = g * _L
                ev = e_v[pl.ds(off, _L)]
                nv = lax.shift_right_logical(base + off + iota, 6)
                pos = 2 * (off + iota)
                plsc.store_scatter(idx_v, [pos], nv)
                plsc.store_scatter(idx_v, [pos + 1], ev)
                mi = plsc.load_gather(m_v, [nv])
                mj = plsc.load_gather(m_v, [ev])
                mk_v[pl.ds(off, _L)] = mi * mj
                return carry2

            lax.fori_loop(0, _CHUNK // _L, grp, 0)

            copies = []
            for j in range(_NSUB):
                copies.append(pltpu.async_copy(
                    x_hbm.at[idx_v.at[pl.ds(j * 128, 128)]],
                    rows_v.at[pl.ds(j * 128, 128)], sem))
            for cp in copies:
                cp.wait()

            pltpu.sync_copy(rows_v, out_hbm.at[pl.ds(2 * base, _ROWS)])
            pltpu.sync_copy(mk_v, mout_hbm.at[pl.ds(base, _CHUNK)])

        return carry

    lax.fori_loop(0, (_NCHUNK + _NW - 1) // _NW, chunk_body, 0)


_edge2grids_sc = functools.partial(
    pl.kernel,
    out_type=(jax.ShapeDtypeStruct((2 * _N * _K, _D), jnp.float32),
              jax.ShapeDtypeStruct((_N * _K,), jnp.float32)),
    mesh=plsc.VectorSubcoreMesh(core_axis_name="c", subcore_axis_name="s",
                                num_cores=_NC, num_subcores=_NS),
    compiler_params=pltpu.CompilerParams(needs_layout_passes=False,
                                         use_tc_tiling_on_sc=False),
    scratch_types=[
        pltpu.VMEM((_N,), jnp.int32),
        pltpu.VMEM((_N,), jnp.float32),
        pltpu.VMEM((_CHUNK,), jnp.int32),
        pltpu.VMEM((_ROWS,), jnp.int32),
        pltpu.VMEM((_ROWS, _D), jnp.float32),
        pltpu.VMEM((_CHUNK,), jnp.float32),
        pltpu.SemaphoreType.DMA,
    ],
)(_edge2grids_body)


def kernel(X, edge_idx, C):
    num_batch, num_residues, num_neighbors = edge_idx.shape
    num_types = X.shape[2]
    assert (num_batch, num_residues, num_neighbors) == (1, _N, _K)
    assert num_types * 3 == _D

    x_flat = X.reshape(_N, _D)
    e_flat = edge_idx.reshape(_N * _K)
    c_flat = C.reshape(_N)

    out_rows, mask = _edge2grids_sc(x_flat, e_flat, c_flat)

    x_ij = out_rows.reshape(num_batch, _N, _K, 2 * num_types, 3)
    mask_ij = mask.reshape(num_batch, _N, _K, 1)
    return (x_ij, mask_ij)
